# baseline (speedup 1.0000x reference)
import functools

import jax
import jax.numpy as jnp
import numpy as np
from jax import lax
from jax.experimental import pallas as pl
from jax.experimental.pallas import tpu as pltpu
from jax.experimental.pallas import tpu_sc as plsc

VOCAB = 100000
D_MODEL = 768
B = 4
S = 4096

_NC = 2
_NS = 16
_NW = _NC * _NS
_P = S // _NW
_C = 16
_NCHUNK = _P // _C
_LANES = 16
_DCH = D_MODEL // _LANES
_JU = 8


def _pos_encoding(seq_len, d_model):
    pos = np.arange(seq_len, dtype=np.float32)[:, None]
    i = np.arange(0, d_model, 2, dtype=np.float32)
    div = np.power(np.float32(10000.0), i / np.float32(d_model))
    pe = np.zeros((seq_len, d_model), dtype=np.float32)
    pe[:, 0::2] = np.sin(pos / div)
    pe[:, 1::2] = np.cos(pos / div)
    return jnp.asarray(pe)


_mesh = plsc.VectorSubcoreMesh(core_axis_name="c", subcore_axis_name="s")


@functools.partial(
    pl.kernel,
    mesh=_mesh,
    out_type=jax.ShapeDtypeStruct((B, S, D_MODEL), jnp.float32),
    scratch_types=[
        pltpu.VMEM((_NCHUNK, B * _C), jnp.int32),
        pltpu.VMEM((2, _C, D_MODEL), jnp.float32),
        pltpu.VMEM((2, B * _C, D_MODEL), jnp.float32),
        pltpu.SemaphoreType.DMA,
        pltpu.SemaphoreType.DMA,
        pltpu.SemaphoreType.DMA,
    ],
)
def _emb_kernel(x_hbm, pe_hbm, table_hbm, out_hbm, idx_all, pe2, rows2,
                gsem, ssem, psem):
    wid = lax.axis_index("s") * _NC + lax.axis_index("c")
    base_pos = wid * _P

    idx_h = [
        pltpu.async_copy(x_hbm.at[b, pl.ds(base_pos + c * _C, _C)],
                         idx_all.at[c, pl.ds(b * _C, _C)], gsem)
        for c in range(_NCHUNK) for b in range(B)
    ]
    for h in idx_h:
        h.wait()

    gather_h = [None, None]
    pe_h = [None, None]
    store_h = [None, None]

    def start_chunk(c, slot):
        pe_h[slot] = pltpu.async_copy(
            pe_hbm.at[pl.ds(base_pos + c * _C, _C)], pe2.at[slot], psem)
        gather_h[slot] = pltpu.async_copy(
            table_hbm.at[idx_all.at[c]], rows2.at[slot], gsem)

    start_chunk(0, 0)

    for c in range(_NCHUNK):
        k = c % 2
        kn = (c + 1) % 2
        if c + 1 < _NCHUNK:
            if store_h[kn] is not None:
                for h in store_h[kn]:
                    h.wait()
            start_chunk(c + 1, kn)
        gather_h[k].wait()
        pe_h[k].wait()

        def _jblock(jb, _):
            @plsc.parallel_loop(0, _C, 1, unroll=2)
            def _add_row(r):
                for jj in range(_JU):
                    sl = pl.ds((jb * _JU + jj) * _LANES, _LANES)
                    p = pe2[k, r, sl]
                    for b in range(B):
                        plsc.addupdate(rows2.at[k, b * _C + r, sl], p)
            return 0

        lax.fori_loop(0, _DCH // _JU, _jblock, 0)

        store_h[k] = [
            pltpu.async_copy(rows2.at[k, pl.ds(b * _C, _C)],
                             out_hbm.at[b, pl.ds(base_pos + c * _C, _C)],
                             ssem)
            for b in range(B)
        ]

    for hs in store_h:
        if hs is not None:
            for h in hs:
                h.wait()


def kernel(x, tok_table):
    pe = _pos_encoding(S, D_MODEL)
    return _emb_kernel(x.astype(jnp.int32), pe, tok_table)


# speedup vs baseline: 1.0183x; 1.0183x over previous
import functools

import jax
import jax.numpy as jnp
import numpy as np
from jax import lax
from jax.experimental import pallas as pl
from jax.experimental.pallas import tpu as pltpu
from jax.experimental.pallas import tpu_sc as plsc

VOCAB = 100000
D_MODEL = 768
B = 4
S = 4096

_NC = 2
_NS = 16
_NW = _NC * _NS
_P = S // _NW
_C = 16
_NCHUNK = _P // _C
_LANES = 16
_DCH = D_MODEL // _LANES
_JU = 8


def _pos_encoding(seq_len, d_model):
    pos = np.arange(seq_len, dtype=np.float32)[:, None]
    i = np.arange(0, d_model, 2, dtype=np.float32)
    div = np.power(np.float32(10000.0), i / np.float32(d_model))
    pe = np.zeros((seq_len, d_model), dtype=np.float32)
    pe[:, 0::2] = np.sin(pos / div)
    pe[:, 1::2] = np.cos(pos / div)
    return jnp.asarray(pe)


_mesh = plsc.VectorSubcoreMesh(core_axis_name="c", subcore_axis_name="s")


@functools.partial(
    pl.kernel,
    mesh=_mesh,
    out_type=jax.ShapeDtypeStruct((B, S, D_MODEL), jnp.float32),
    scratch_types=[
        pltpu.VMEM((_NCHUNK, B * _C), jnp.int32),
        pltpu.VMEM((2, _C, D_MODEL), jnp.float32),
        pltpu.VMEM((2, B * _C, D_MODEL), jnp.float32),
        pltpu.SemaphoreType.DMA,
        pltpu.SemaphoreType.DMA,
        pltpu.SemaphoreType.DMA,
        pltpu.SemaphoreType.DMA,
    ],
)
def _emb_kernel(x_hbm, pe_hbm, table_hbm, out_hbm, idx_all, pe2, rows2,
                gsem, ssem, psem, isem):
    wid = lax.axis_index("s") * _NC + lax.axis_index("c")
    base_pos = wid * _P

    idx_h = [
        [pltpu.async_copy(x_hbm.at[b, pl.ds(base_pos + c * _C, _C)],
                          idx_all.at[c, pl.ds(b * _C, _C)], isem)
         for b in range(B)]
        for c in range(_NCHUNK)
    ]
    for h in idx_h[0]:
        h.wait()

    gather_h = [None, None]
    pe_h = [None, None]
    store_h = [None, None]

    def start_chunk(c, slot):
        pe_h[slot] = pltpu.async_copy(
            pe_hbm.at[pl.ds(base_pos + c * _C, _C)], pe2.at[slot], psem)
        gather_h[slot] = pltpu.async_copy(
            table_hbm.at[idx_all.at[c]], rows2.at[slot], gsem)

    start_chunk(0, 0)

    for c in range(_NCHUNK):
        k = c % 2
        kn = (c + 1) % 2
        if c + 1 < _NCHUNK:
            for h in idx_h[c + 1]:
                h.wait()
            if store_h[kn] is not None:
                for h in store_h[kn]:
                    h.wait()
            start_chunk(c + 1, kn)
        gather_h[k].wait()
        pe_h[k].wait()

        def _jblock(jb, _):
            @plsc.parallel_loop(0, _C, 1, unroll=1)
            def _add_row(r):
                for jj in range(_JU):
                    sl = pl.ds((jb * _JU + jj) * _LANES, _LANES)
                    p = pe2[k, r, sl]
                    for b in range(B):
                        plsc.addupdate(rows2.at[k, b * _C + r, sl], p)
            return 0

        lax.fori_loop(0, _DCH // _JU, _jblock, 0)

        store_h[k] = [
            pltpu.async_copy(rows2.at[k, pl.ds(b * _C, _C)],
                             out_hbm.at[b, pl.ds(base_pos + c * _C, _C)],
                             ssem)
            for b in range(B)
        ]

    for hs in store_h:
        if hs is not None:
            for h in hs:
                h.wait()


def kernel(x, tok_table):
    pe = _pos_encoding(S, D_MODEL)
    return _emb_kernel(x.astype(jnp.int32), pe, tok_table)


# speedup vs baseline: 1.0386x; 1.0200x over previous
import functools

import jax
import jax.numpy as jnp
import numpy as np
from jax import lax
from jax.experimental import pallas as pl
from jax.experimental.pallas import tpu as pltpu
from jax.experimental.pallas import tpu_sc as plsc

VOCAB = 100000
D_MODEL = 768
B = 4
S = 4096

_NC = 2
_NS = 16
_NW = _NC * _NS
_P = S // _NW
_C = 16
_SCHED = (8, 16, 16, 16, 16, 16, 16, 16, 8)
_OFFS = tuple(sum(_SCHED[:i]) for i in range(len(_SCHED)))
_NCHUNK = len(_SCHED)
_LANES = 16
_DCH = D_MODEL // _LANES
_JU = 8


def _pos_encoding(seq_len, d_model):
    pos = np.arange(seq_len, dtype=np.float32)[:, None]
    i = np.arange(0, d_model, 2, dtype=np.float32)
    div = np.power(np.float32(10000.0), i / np.float32(d_model))
    pe = np.zeros((seq_len, d_model), dtype=np.float32)
    pe[:, 0::2] = np.sin(pos / div)
    pe[:, 1::2] = np.cos(pos / div)
    return jnp.asarray(pe)


_mesh = plsc.VectorSubcoreMesh(core_axis_name="c", subcore_axis_name="s")


@functools.partial(
    pl.kernel,
    mesh=_mesh,
    out_type=jax.ShapeDtypeStruct((B, S, D_MODEL), jnp.float32),
    scratch_types=[
        pltpu.VMEM((-(-_NCHUNK // 4) * 4, B * _C), jnp.int32),
        pltpu.VMEM((2, _C, D_MODEL), jnp.float32),
        pltpu.VMEM((2, B * _C, D_MODEL), jnp.float32),
        pltpu.SemaphoreType.DMA,
        pltpu.SemaphoreType.DMA,
        pltpu.SemaphoreType.DMA,
        pltpu.SemaphoreType.DMA,
    ],
)
def _emb_kernel(x_hbm, pe_hbm, table_hbm, out_hbm, idx_all, pe2, rows2,
                gsem, ssem, psem, isem):
    wid = lax.axis_index("s") * _NC + lax.axis_index("c")
    base_pos = wid * _P

    idx_h = [
        [pltpu.async_copy(
            x_hbm.at[pl.ds(b * S + base_pos + _OFFS[c], _SCHED[c])],
            idx_all.at[c, pl.ds(b * _SCHED[c], _SCHED[c])], isem)
         for b in range(B)]
        for c in range(_NCHUNK)
    ]
    for h in idx_h[0]:
        h.wait()

    gather_h = [None, None]
    pe_h = [None, None]
    store_h = [None, None]

    def start_chunk(c, slot):
        n = _SCHED[c]
        pe_h[slot] = pltpu.async_copy(
            pe_hbm.at[pl.ds(base_pos + _OFFS[c], n)],
            pe2.at[slot, pl.ds(0, n)], psem)
        gather_h[slot] = pltpu.async_copy(
            table_hbm.at[idx_all.at[c, pl.ds(0, B * n)]],
            rows2.at[slot, pl.ds(0, B * n)], gsem)

    start_chunk(0, 0)

    for c in range(_NCHUNK):
        n = _SCHED[c]
        k = c % 2
        kn = (c + 1) % 2
        if c + 1 < _NCHUNK:
            for h in idx_h[c + 1]:
                h.wait()
            if store_h[kn] is not None:
                for h in store_h[kn]:
                    h.wait()
            start_chunk(c + 1, kn)
        gather_h[k].wait()
        pe_h[k].wait()

        def _jblock(jb, _):
            @plsc.parallel_loop(0, n, 1, unroll=1)
            def _add_row(r):
                for jj in range(_JU):
                    sl = pl.ds((jb * _JU + jj) * _LANES, _LANES)
                    p = pe2[k, r, sl]
                    for b in range(B):
                        plsc.addupdate(rows2.at[k, b * n + r, sl], p)
            return 0

        lax.fori_loop(0, _DCH // _JU, _jblock, 0)

        store_h[k] = [
            pltpu.async_copy(rows2.at[k, pl.ds(b * n, n)],
                             out_hbm.at[b, pl.ds(base_pos + _OFFS[c], n)],
                             ssem)
            for b in range(B)
        ]

    for hs in store_h:
        if hs is not None:
            for h in hs:
                h.wait()


def kernel(x, tok_table):
    pe = _pos_encoding(S, D_MODEL)
    return _emb_kernel(x.astype(jnp.int32).reshape(B * S), pe, tok_table)
